# baseline (speedup 1.0000x reference)
import jax
import jax.numpy as jnp
from jax.experimental import pallas as pl
from jax.experimental.pallas import tpu as pltpu

_BF = jnp.bfloat16


def _moe_block(xb, xb_bf, wg_ref, wt_s, cs_s):
    logits = jnp.dot(xb, wg_ref[...], preferred_element_type=jnp.float32)
    p = jnp.exp(logits)
    p_bf = p.astype(_BF)
    e_num = p.shape[1]
    d = xb.shape[1]
    zu = jnp.concatenate([p_bf[:, e:e + 1] * xb_bf for e in range(e_num)],
                         axis=1)
    rr = jnp.dot(p, cs_s[...], preferred_element_type=jnp.float32)
    hu = (jnp.dot(zu, wt_s[...], preferred_element_type=jnp.float32)
          + rr[:, :d])
    return hu * (1.0 / rr[:, d:])


def _fused_kernel(x_ref, wgA_ref, weA_ref, beA_ref, wgB_ref, weB_ref,
                  beB_ref, out_ref, wtA_s, csA_s, wtB_s, csB_s):
    @pl.when(pl.program_id(0) == 0)
    def _prologue():
        for we_ref, be_ref, wt_s, cs_s in (
                (weA_ref, beA_ref, wtA_s, csA_s),
                (weB_ref, beB_ref, wtB_s, csB_s)):
            w = we_ref[...]
            e_num, d_num = w.shape[0], w.shape[1]
            for e in range(e_num):
                wt_s[e * d_num:(e + 1) * d_num, :] = (
                    jnp.transpose(w[e], (1, 0)).astype(_BF))
            c = -jnp.sum(be_ref[...][:, None, :] * w, axis=-1)
            cs_s[...] = jnp.concatenate(
                [c, jnp.ones_like(c)], axis=1)

    xb = x_ref[...]
    h = _moe_block(xb, xb.astype(_BF), wgA_ref, wtA_s, csA_s)
    o = _moe_block(h, h.astype(_BF), wgB_ref, wtB_s, csB_s)
    out_ref[...] = o + xb


def kernel(x, wgA, WeA, beA, wgB, WeB, beB):
    t, d = x.shape
    e = wgA.shape[1]
    ed = e * d
    tb = 1024
    grid = (t // tb,)
    full = lambda shape: pl.BlockSpec(shape, lambda i: tuple(0 for _ in shape))
    return pl.pallas_call(
        _fused_kernel,
        grid=grid,
        in_specs=[
            pl.BlockSpec((tb, d), lambda i: (i, 0)),
            full((d, e)), full((e, d, d)), full((e, d)),
            full((d, e)), full((e, d, d)), full((e, d)),
        ],
        out_specs=pl.BlockSpec((tb, d), lambda i: (i, 0)),
        out_shape=jax.ShapeDtypeStruct((t, d), x.dtype),
        scratch_shapes=[
            pltpu.VMEM((ed, d), _BF), pltpu.VMEM((e, 2 * d), jnp.float32),
            pltpu.VMEM((ed, d), _BF), pltpu.VMEM((e, 2 * d), jnp.float32),
        ],
    )(x, wgA, WeA, beA, wgB, WeB, beB)


# speedup vs baseline: 1.2850x; 1.2850x over previous
import jax
import jax.numpy as jnp
from jax.experimental import pallas as pl
from jax.experimental.pallas import tpu as pltpu

_BF = jnp.bfloat16


def _moe_block(xb_bf, wg_s, wt_s, cs_s, erep_ref):
    logits = jnp.dot(xb_bf, wg_s[...], preferred_element_type=jnp.float32)
    p_bf = jnp.exp(logits).astype(_BF)
    d = xb_bf.shape[1]
    p_exp = jnp.dot(p_bf, erep_ref[...],
                    preferred_element_type=jnp.float32).astype(_BF)
    ed = wt_s.shape[0]
    xrep = jnp.concatenate([xb_bf] * (ed // d), axis=1)
    zu = p_exp * xrep
    rr = jnp.dot(p_bf, cs_s[...], preferred_element_type=jnp.float32)
    hu = (jnp.dot(zu, wt_s[...], preferred_element_type=jnp.float32)
          + rr[:, :d])
    return hu * (1.0 / rr[:, d:])


def _fused_kernel(x_ref, wgA_ref, weA_ref, beA_ref, wgB_ref, weB_ref,
                  beB_ref, erep_ref, out_ref,
                  wgA_s, wtA_s, csA_s, wgB_s, wtB_s, csB_s):
    @pl.when(pl.program_id(0) == 0)
    def _prologue():
        for wg_ref, we_ref, be_ref, wg_s, wt_s, cs_s in (
                (wgA_ref, weA_ref, beA_ref, wgA_s, wtA_s, csA_s),
                (wgB_ref, weB_ref, beB_ref, wgB_s, wtB_s, csB_s)):
            w = we_ref[...]
            e_num, d_num = w.shape[0], w.shape[1]
            for e in range(e_num):
                wt_s[e * d_num:(e + 1) * d_num, :] = (
                    jnp.transpose(w[e], (1, 0)).astype(_BF))
            c = -jnp.sum(be_ref[...][:, None, :] * w, axis=-1)
            cs_s[...] = jnp.concatenate(
                [c, jnp.ones_like(c)], axis=1).astype(_BF)
            wg_s[...] = wg_ref[...].astype(_BF)

    xb = x_ref[...]
    h = _moe_block(xb.astype(_BF), wgA_s, wtA_s, csA_s, erep_ref)
    o = _moe_block(h.astype(_BF), wgB_s, wtB_s, csB_s, erep_ref)
    out_ref[...] = o + xb


def kernel(x, wgA, WeA, beA, wgB, WeB, beB):
    t, d = x.shape
    e = wgA.shape[1]
    ed = e * d
    erep = jnp.repeat(jnp.eye(e, dtype=_BF), d, axis=1)

    tb = 1024
    grid = (t // tb,)
    full = lambda shape: pl.BlockSpec(shape, lambda i: tuple(0 for _ in shape))
    layer_scratch = [pltpu.VMEM((d, e), _BF), pltpu.VMEM((ed, d), _BF),
                     pltpu.VMEM((e, 2 * d), _BF)]
    return pl.pallas_call(
        _fused_kernel,
        grid=grid,
        in_specs=[
            pl.BlockSpec((tb, d), lambda i: (i, 0)),
            full((d, e)), full((e, d, d)), full((e, d)),
            full((d, e)), full((e, d, d)), full((e, d)),
            full((e, ed)),
        ],
        out_specs=pl.BlockSpec((tb, d), lambda i: (i, 0)),
        out_shape=jax.ShapeDtypeStruct((t, d), x.dtype),
        scratch_shapes=layer_scratch + layer_scratch,
    )(x, wgA, WeA, beA, wgB, WeB, beB, erep)


# speedup vs baseline: 1.4168x; 1.1025x over previous
import jax
import jax.numpy as jnp
from jax.experimental import pallas as pl
from jax.experimental.pallas import tpu as pltpu

_BF = jnp.bfloat16


def _moe_block(xb_bf, wg_s, wt_s, cs_s, erep_ref):
    logits = jnp.dot(xb_bf, wg_s[...], preferred_element_type=jnp.float32)
    p_bf = jnp.exp(logits).astype(_BF)
    d = xb_bf.shape[1]
    p_exp = jnp.dot(p_bf, erep_ref[...],
                    preferred_element_type=jnp.float32).astype(_BF)
    ed = wt_s.shape[0]
    xrep = jnp.concatenate([xb_bf] * (ed // d), axis=1)
    zu = p_exp * xrep
    rr = jnp.dot(p_bf, cs_s[...], preferred_element_type=jnp.float32)
    hu = (jnp.dot(zu, wt_s[...], preferred_element_type=jnp.float32)
          + rr[:, :d])
    return hu * (1.0 / rr[:, d:])


def _fused_kernel(x_ref, wgA_ref, weA_ref, beA_ref, wgB_ref, weB_ref,
                  beB_ref, erep_ref, out_ref,
                  wgA_s, wtA_s, csA_s, wgB_s, wtB_s, csB_s):
    @pl.when(pl.program_id(0) == 0)
    def _prologue():
        for wg_ref, we_ref, be_ref, wg_s, wt_s, cs_s in (
                (wgA_ref, weA_ref, beA_ref, wgA_s, wtA_s, csA_s),
                (wgB_ref, weB_ref, beB_ref, wgB_s, wtB_s, csB_s)):
            w = we_ref[...]
            e_num, d_num = w.shape[0], w.shape[1]
            for e in range(e_num):
                wt_s[e * d_num:(e + 1) * d_num, :] = (
                    jnp.transpose(w[e], (1, 0)).astype(_BF))
            c = -jnp.sum(be_ref[...][:, None, :] * w, axis=-1)
            cs_s[...] = jnp.concatenate(
                [c, jnp.ones_like(c)], axis=1).astype(_BF)
            wg_s[...] = wg_ref[...].astype(_BF)

    xb = x_ref[...]
    h = _moe_block(xb.astype(_BF), wgA_s, wtA_s, csA_s, erep_ref)
    o = _moe_block(h.astype(_BF), wgB_s, wtB_s, csB_s, erep_ref)
    out_ref[...] = o + xb


def kernel(x, wgA, WeA, beA, wgB, WeB, beB):
    t, d = x.shape
    e = wgA.shape[1]
    ed = e * d
    erep = jnp.repeat(jnp.eye(e, dtype=_BF), d, axis=1)

    tb = 2048
    grid = (t // tb,)
    full = lambda shape: pl.BlockSpec(shape, lambda i: tuple(0 for _ in shape))
    layer_scratch = [pltpu.VMEM((d, e), _BF), pltpu.VMEM((ed, d), _BF),
                     pltpu.VMEM((e, 2 * d), _BF)]
    return pl.pallas_call(
        _fused_kernel,
        grid=grid,
        in_specs=[
            pl.BlockSpec((tb, d), lambda i: (i, 0)),
            full((d, e)), full((e, d, d)), full((e, d)),
            full((d, e)), full((e, d, d)), full((e, d)),
            full((e, ed)),
        ],
        out_specs=pl.BlockSpec((tb, d), lambda i: (i, 0)),
        out_shape=jax.ShapeDtypeStruct((t, d), x.dtype),
        scratch_shapes=layer_scratch + layer_scratch,
    )(x, wgA, WeA, beA, wgB, WeB, beB, erep)


# speedup vs baseline: 1.4436x; 1.0190x over previous
import jax
import jax.numpy as jnp
from jax.experimental import pallas as pl
from jax.experimental.pallas import tpu as pltpu

_BF = jnp.bfloat16


def _moe_block(xb_bf, wg_s, wt_s, cs_s, erep_ref):
    logits = jnp.dot(xb_bf, wg_s[...], preferred_element_type=jnp.float32)
    p_bf = jnp.exp(logits).astype(_BF)
    d = xb_bf.shape[1]
    p_exp = jnp.dot(p_bf, erep_ref[...],
                    preferred_element_type=jnp.float32).astype(_BF)
    ed = wt_s.shape[0]
    xrep = jnp.concatenate([xb_bf] * (ed // d), axis=1)
    zu = p_exp * xrep
    rr = jnp.dot(p_bf, cs_s[...], preferred_element_type=jnp.float32)
    hu = (jnp.dot(zu, wt_s[...], preferred_element_type=jnp.float32)
          + rr[:, :d])
    return hu * (1.0 / rr[:, d:])


def _fused_kernel(x_ref, wgA_ref, weA_ref, beA_ref, wgB_ref, weB_ref,
                  beB_ref, erep_ref, out_ref,
                  wgA_s, wtA_s, csA_s, wgB_s, wtB_s, csB_s):
    @pl.when(pl.program_id(0) == 0)
    def _prologue():
        for wg_ref, we_ref, be_ref, wg_s, wt_s, cs_s in (
                (wgA_ref, weA_ref, beA_ref, wgA_s, wtA_s, csA_s),
                (wgB_ref, weB_ref, beB_ref, wgB_s, wtB_s, csB_s)):
            w = we_ref[...]
            e_num, d_num = w.shape[0], w.shape[1]
            for e in range(e_num):
                wt_s[e * d_num:(e + 1) * d_num, :] = (
                    jnp.transpose(w[e], (1, 0)).astype(_BF))
            c = -jnp.sum(be_ref[...][:, None, :] * w, axis=-1)
            cs_s[...] = jnp.concatenate(
                [c, jnp.ones_like(c)], axis=1).astype(_BF)
            wg_s[...] = wg_ref[...].astype(_BF)

    xb = x_ref[...]
    h = _moe_block(xb.astype(_BF), wgA_s, wtA_s, csA_s, erep_ref)
    o = _moe_block(h.astype(_BF), wgB_s, wtB_s, csB_s, erep_ref)
    out_ref[...] = o + xb


def kernel(x, wgA, WeA, beA, wgB, WeB, beB):
    t, d = x.shape
    e = wgA.shape[1]
    ed = e * d
    erep = jnp.repeat(jnp.eye(e, dtype=_BF), d, axis=1)

    tb = 4096
    grid = (t // tb,)
    full = lambda shape: pl.BlockSpec(shape, lambda i: tuple(0 for _ in shape))
    layer_scratch = [pltpu.VMEM((d, e), _BF), pltpu.VMEM((ed, d), _BF),
                     pltpu.VMEM((e, 2 * d), _BF)]
    return pl.pallas_call(
        _fused_kernel,
        grid=grid,
        in_specs=[
            pl.BlockSpec((tb, d), lambda i: (i, 0)),
            full((d, e)), full((e, d, d)), full((e, d)),
            full((d, e)), full((e, d, d)), full((e, d)),
            full((e, ed)),
        ],
        out_specs=pl.BlockSpec((tb, d), lambda i: (i, 0)),
        out_shape=jax.ShapeDtypeStruct((t, d), x.dtype),
        scratch_shapes=layer_scratch + layer_scratch,
    )(x, wgA, WeA, beA, wgB, WeB, beB, erep)


# speedup vs baseline: 2.1158x; 1.4656x over previous
import jax
import jax.numpy as jnp
from jax.experimental import pallas as pl
from jax.experimental.pallas import tpu as pltpu

_BF = jnp.bfloat16


def _moe_block_t(hT, hT_bf, wgT_s, w2_s, ct_s):
    e_num = wgT_s.shape[0]
    lgT = jnp.dot(wgT_s[...], hT_bf, preferred_element_type=jnp.float32)
    pT = jnp.exp(lgT)
    pT_bf = pT.astype(_BF)
    rinvT = 1.0 / jnp.sum(pT, axis=0, keepdims=True)
    d, tb = hT.shape
    pT_rep = jnp.broadcast_to(pT_bf[:, None, :], (e_num, d, tb)
                              ).reshape(e_num * d, tb)
    hT_rep = jnp.concatenate([hT_bf] * e_num, axis=0)
    zuT = pT_rep * hT_rep
    huT = (jnp.dot(w2_s[...], zuT, preferred_element_type=jnp.float32)
           + jnp.dot(ct_s[...], pT_bf, preferred_element_type=jnp.float32))
    return huT * rinvT


def _fused_kernel(x_ref, wgA_ref, weA_ref, beA_ref, wgB_ref, weB_ref,
                  beB_ref, out_ref,
                  wgA_s, w2A_s, ctA_s, wgB_s, w2B_s, ctB_s):
    @pl.when(pl.program_id(0) == 0)
    def _prologue():
        for wg_ref, we_ref, be_ref, wg_s, w2_s, ct_s in (
                (wgA_ref, weA_ref, beA_ref, wgA_s, w2A_s, ctA_s),
                (wgB_ref, weB_ref, beB_ref, wgB_s, w2B_s, ctB_s)):
            w = we_ref[...]
            e_num, d_num = w.shape[0], w.shape[1]
            w2_s[...] = jnp.transpose(w, (1, 0, 2)).reshape(
                d_num, e_num * d_num).astype(_BF)
            c = -jnp.sum(be_ref[...][:, None, :] * w, axis=-1)
            ct_s[...] = jnp.transpose(c, (1, 0)).astype(_BF)
            wg_s[...] = jnp.transpose(wg_ref[...], (1, 0)).astype(_BF)

    xb = x_ref[...]
    xT = jnp.transpose(xb, (1, 0))
    hT = _moe_block_t(xT, xT.astype(_BF), wgA_s, w2A_s, ctA_s)
    oT = _moe_block_t(hT, hT.astype(_BF), wgB_s, w2B_s, ctB_s)
    out_ref[...] = jnp.transpose(oT + xT, (1, 0))


def kernel(x, wgA, WeA, beA, wgB, WeB, beB):
    t, d = x.shape
    e = wgA.shape[1]
    ed = e * d

    tb = 4096
    grid = (t // tb,)
    full = lambda shape: pl.BlockSpec(shape, lambda i: tuple(0 for _ in shape))
    layer_scratch = [pltpu.VMEM((e, d), _BF), pltpu.VMEM((d, ed), _BF),
                     pltpu.VMEM((d, e), _BF)]
    return pl.pallas_call(
        _fused_kernel,
        grid=grid,
        in_specs=[
            pl.BlockSpec((tb, d), lambda i: (i, 0)),
            full((d, e)), full((e, d, d)), full((e, d)),
            full((d, e)), full((e, d, d)), full((e, d)),
        ],
        out_specs=pl.BlockSpec((tb, d), lambda i: (i, 0)),
        out_shape=jax.ShapeDtypeStruct((t, d), x.dtype),
        scratch_shapes=layer_scratch + layer_scratch,
    )(x, wgA, WeA, beA, wgB, WeB, beB)
